# Initial kernel scaffold; baseline (speedup 1.0000x reference)
import jax
import jax.numpy as jnp
from jax.experimental import pallas as pl


def kernel(input, points, support_points, indices):
    raise NotImplementedError("write your pallas kernel here")


# speedup vs baseline: 3928.6948x; 3928.6948x over previous
import functools

import jax
import jax.numpy as jnp
from jax import lax
from jax.experimental import pallas as pl
from jax.experimental.pallas import tpu as pltpu
from jax.experimental.pallas import tpu_sc as plsc

_NC, _NS, _L = 2, 16, 16
_NW = _NC * _NS


@functools.lru_cache(maxsize=None)
def _build(B, C, N, M, K):
    rows_per_w = (B * C) // _NW
    w_per_b = C // rows_per_w
    assert rows_per_w * _NW == B * C and w_per_b * rows_per_w == C

    mesh = plsc.VectorSubcoreMesh(
        core_axis_name="c", subcore_axis_name="s",
        num_cores=_NC, num_subcores=_NS)

    @functools.partial(
        pl.kernel,
        out_type=jax.ShapeDtypeStruct((B * C, M), jnp.float32),
        mesh=mesh,
        compiler_params=pltpu.CompilerParams(needs_layout_passes=False),
        scratch_types=[
            pltpu.VMEM((K, M), jnp.int32),
            pltpu.VMEM((N,), jnp.float32),
            pltpu.VMEM((M,), jnp.float32),
        ],
    )
    def gather_max(feat_hbm, idx_hbm, out_hbm, idx_v, row_v, out_v):
        wid = lax.axis_index("s") * _NC + lax.axis_index("c")
        b = wid // w_per_b
        pltpu.sync_copy(idx_hbm.at[b], idx_v)

        for j in range(rows_per_w):
            r = wid * rows_per_w + j
            pltpu.sync_copy(feat_hbm.at[r], row_v)

            def mg_body(mg, _):
                base = pl.multiple_of(mg * _L, _L)
                acc = plsc.load_gather(row_v, [idx_v[0, pl.ds(base, _L)]])
                for kk in range(1, K):
                    g = plsc.load_gather(row_v, [idx_v[kk, pl.ds(base, _L)]])
                    acc = jnp.maximum(acc, g)
                out_v[pl.ds(base, _L)] = acc
                return 0

            lax.fori_loop(0, M // _L, mg_body, 0)
            pltpu.sync_copy(out_v, out_hbm.at[r])

    return gather_max


def kernel(input, points, support_points, indices):
    del points, support_points
    B, C, N = input.shape
    _, M, K = indices.shape
    feat = input.reshape(B * C, N)
    idx_t = indices.astype(jnp.int32).transpose(0, 2, 1)
    out = _build(B, C, N, M, K)(feat, idx_t)
    return out.reshape(B, C, M)


# speedup vs baseline: 4749.9115x; 1.2090x over previous
import functools

import jax
import jax.numpy as jnp
from jax import lax
from jax.experimental import pallas as pl
from jax.experimental.pallas import tpu as pltpu
from jax.experimental.pallas import tpu_sc as plsc

_NC, _NS, _L = 2, 16, 16
_NW = _NC * _NS


@functools.lru_cache(maxsize=None)
def _build(B, C, N, M, K):
    rows_per_w = (B * C) // _NW
    w_per_b = C // rows_per_w
    assert rows_per_w * _NW == B * C and w_per_b * rows_per_w == C

    mesh = plsc.VectorSubcoreMesh(
        core_axis_name="c", subcore_axis_name="s",
        num_cores=_NC, num_subcores=_NS)

    @functools.partial(
        pl.kernel,
        out_type=jax.ShapeDtypeStruct((B * C, M), jnp.float32),
        mesh=mesh,
        compiler_params=pltpu.CompilerParams(needs_layout_passes=False),
        scratch_types=[
            pltpu.VMEM((K, M), jnp.int32),
            pltpu.VMEM((N,), jnp.float32),
            pltpu.VMEM((N,), jnp.float32),
            pltpu.VMEM((M,), jnp.float32),
            pltpu.VMEM((M,), jnp.float32),
        ],
    )
    def gather_max(feat_hbm, idx_hbm, out_hbm, idx_v, row0_v, row1_v,
                   out0_v, out1_v):
        wid = lax.axis_index("s") * _NC + lax.axis_index("c")
        b = wid // w_per_b
        pltpu.sync_copy(idx_hbm.at[b], idx_v)

        for j in range(rows_per_w // 2):
            r = wid * rows_per_w + 2 * j
            pltpu.sync_copy(feat_hbm.at[r], row0_v)
            pltpu.sync_copy(feat_hbm.at[r + 1], row1_v)

            def mg_body(mg, _):
                base = pl.multiple_of(mg * _L, _L)
                iv = [idx_v[kk, pl.ds(base, _L)] for kk in range(K)]
                acc0 = plsc.load_gather(row0_v, [iv[0]])
                acc1 = plsc.load_gather(row1_v, [iv[0]])
                for kk in range(1, K):
                    acc0 = jnp.maximum(acc0, plsc.load_gather(row0_v, [iv[kk]]))
                    acc1 = jnp.maximum(acc1, plsc.load_gather(row1_v, [iv[kk]]))
                out0_v[pl.ds(base, _L)] = acc0
                out1_v[pl.ds(base, _L)] = acc1
                return 0

            lax.fori_loop(0, M // _L, mg_body, 0)
            pltpu.sync_copy(out0_v, out_hbm.at[r])
            pltpu.sync_copy(out1_v, out_hbm.at[r + 1])

    return gather_max


def kernel(input, points, support_points, indices):
    del points, support_points
    B, C, N = input.shape
    _, M, K = indices.shape
    feat = input.reshape(B * C, N)
    idx_t = indices.astype(jnp.int32).transpose(0, 2, 1)
    out = _build(B, C, N, M, K)(feat, idx_t)
    return out.reshape(B, C, M)


# speedup vs baseline: 4837.6277x; 1.0185x over previous
import functools

import jax
import jax.numpy as jnp
from jax import lax
from jax.experimental import pallas as pl
from jax.experimental.pallas import tpu as pltpu
from jax.experimental.pallas import tpu_sc as plsc

_NC, _NS, _L = 2, 16, 16
_NW = _NC * _NS


@functools.lru_cache(maxsize=None)
def _build(B, C, N, M, K):
    rows_per_w = (B * C) // _NW
    w_per_b = C // rows_per_w
    assert rows_per_w * _NW == B * C and w_per_b * rows_per_w == C

    mesh = plsc.VectorSubcoreMesh(
        core_axis_name="c", subcore_axis_name="s",
        num_cores=_NC, num_subcores=_NS)

    @functools.partial(
        pl.kernel,
        out_type=jax.ShapeDtypeStruct((B * C, M), jnp.float32),
        mesh=mesh,
        compiler_params=pltpu.CompilerParams(needs_layout_passes=False),
        scratch_types=[
            pltpu.VMEM((K // 2, M), jnp.int32),
            pltpu.VMEM((N,), jnp.float32),
            pltpu.VMEM((N,), jnp.float32),
            pltpu.VMEM((M,), jnp.float32),
            pltpu.VMEM((M,), jnp.float32),
        ],
    )
    def gather_max(feat_hbm, idx_hbm, out_hbm, idx_v, row0_v, row1_v,
                   out0_v, out1_v):
        wid = lax.axis_index("s") * _NC + lax.axis_index("c")
        b = wid // w_per_b
        pltpu.sync_copy(idx_hbm.at[b], idx_v)

        for j in range(rows_per_w // 2):
            r = wid * rows_per_w + 2 * j
            pltpu.sync_copy(feat_hbm.at[r], row0_v)
            pltpu.sync_copy(feat_hbm.at[r + 1], row1_v)

            def mg_body(mg, _):
                base = pl.multiple_of(mg * _L, _L)
                iv = []
                for p in range(K // 2):
                    w = idx_v[p, pl.ds(base, _L)]
                    iv.append(jnp.bitwise_and(w, 0xFFFF))
                    iv.append(lax.shift_right_logical(w, 16))
                acc0 = plsc.load_gather(row0_v, [iv[0]])
                acc1 = plsc.load_gather(row1_v, [iv[0]])
                for kk in range(1, K):
                    acc0 = jnp.maximum(acc0, plsc.load_gather(row0_v, [iv[kk]]))
                    acc1 = jnp.maximum(acc1, plsc.load_gather(row1_v, [iv[kk]]))
                out0_v[pl.ds(base, _L)] = acc0
                out1_v[pl.ds(base, _L)] = acc1
                return 0

            lax.fori_loop(0, M // _L, mg_body, 0)
            pltpu.sync_copy(out0_v, out_hbm.at[r])
            pltpu.sync_copy(out1_v, out_hbm.at[r + 1])

    return gather_max


def kernel(input, points, support_points, indices):
    del points, support_points
    B, C, N = input.shape
    _, M, K = indices.shape
    feat = input.reshape(B * C, N)
    idx_t = indices.astype(jnp.int32).transpose(0, 2, 1)
    idx_p = idx_t[:, 0::2, :] | (idx_t[:, 1::2, :] << 16)
    out = _build(B, C, N, M, K)(feat, idx_p)
    return out.reshape(B, C, M)
